# baseline (speedup 1.0000x reference)
import functools

import jax
import jax.numpy as jnp
from jax import lax
from jax.experimental import pallas as pl
from jax.experimental.pallas import tpu as pltpu
from jax.experimental.pallas import tpu_sc as plsc


@functools.cache
def _sc_row_copy(n_elems: int):
    info = plsc.get_sparse_core_info()
    nc = info.num_cores
    assert n_elems % nc == 0
    per_c = n_elems // nc
    assert per_c % 8 == 0

    mesh = plsc.ScalarSubcoreMesh(axis_name="c", num_cores=nc)

    @functools.partial(
        pl.kernel,
        out_type=jax.ShapeDtypeStruct((n_elems,), jnp.float32),
        mesh=mesh,
    )
    def copy_kernel(tab_hbm, out_hbm):
        cid = lax.axis_index("c")
        base = cid * per_c
        pltpu.sync_copy(
            tab_hbm.at[pl.ds(base, per_c)], out_hbm.at[pl.ds(base, per_c)]
        )

    return copy_kernel


def kernel(x, wpe):
    seq_len = x.shape[1]
    emb_dim = wpe.shape[1]
    flat = jnp.reshape(wpe, (-1,))
    out = _sc_row_copy(seq_len * emb_dim)(flat)
    return jnp.reshape(out, (1, seq_len, emb_dim))


# speedup vs baseline: 13.3982x; 13.3982x over previous
import functools

import jax
import jax.numpy as jnp
from jax.experimental import pallas as pl


def _copy_body(wpe_ref, o_ref):
    o_ref[...] = wpe_ref[...]


@functools.cache
def _tc_copy(seq_len: int, emb_dim: int):
    return pl.pallas_call(
        _copy_body,
        out_shape=jax.ShapeDtypeStruct((seq_len, emb_dim), jnp.float32),
    )


def kernel(x, wpe):
    seq_len = x.shape[1]
    emb_dim = wpe.shape[1]
    out = _tc_copy(seq_len, emb_dim)(wpe[:seq_len])
    return jnp.reshape(out, (1, seq_len, emb_dim))
